# baseline (speedup 1.0000x reference)
import functools

import jax
import jax.numpy as jnp
from jax import lax
from jax.experimental import pallas as pl
from jax.experimental.pallas import tpu as pltpu
from jax.experimental.pallas import tpu_sc as plsc

C = 256
N = 10000
E = 160000

NC, NS, L = 2, 16, 16
NW = NC * NS
CHUNK = E // NW
FULL = CHUNK // L

GRID = 2
NBLK = 5120
EBLK = 81920


def _tc_scores(w_ref, z_ref, b_ref, eli_ref, s_ref, src_ref, dst_ref):
    w = w_ref[...]
    w12 = jnp.concatenate([w[:, :C], w[:, C:]], axis=0)
    s = lax.dot_general(
        w12, z_ref[...],
        dimension_numbers=(((1,), (1,)), ((), ())),
        preferred_element_type=jnp.float32,
    )
    row = lax.broadcasted_iota(jnp.int32, (2, NBLK), 0)
    s_ref[...] = s + jnp.where(row == 0, b_ref[0], 0.0)
    src_ref[...] = eli_ref[0]
    dst_ref[...] = eli_ref[1]


_sc_mesh = plsc.VectorSubcoreMesh(core_axis_name="c", subcore_axis_name="s")


@functools.partial(
    pl.kernel,
    out_type=jax.ShapeDtypeStruct((E,), jnp.float32),
    mesh=_sc_mesh,
    scratch_types=[
        pltpu.VMEM((N,), jnp.float32),
        pltpu.VMEM((N,), jnp.float32),
        pltpu.VMEM((CHUNK,), jnp.int32),
        pltpu.VMEM((CHUNK,), jnp.int32),
        pltpu.VMEM((CHUNK,), jnp.float32),
        pltpu.SemaphoreType.DMA,
        pltpu.SemaphoreType.DMA,
        pltpu.SemaphoreType.DMA,
        pltpu.SemaphoreType.DMA,
    ],
    compiler_params=pltpu.CompilerParams(needs_layout_passes=False),
)
def _sc_edge_scores(s_hbm, src_hbm, dst_hbm, out_hbm,
                    s1_v, s2_v, src_v, dst_v, out_v, sem1, sem2, sem3, sem4):
    wid = lax.axis_index("s") * NC + lax.axis_index("c")
    base = wid * CHUNK

    cp1 = pltpu.async_copy(s_hbm.at[0], s1_v, sem1)
    cp2 = pltpu.async_copy(s_hbm.at[1], s2_v, sem2)
    cp3 = pltpu.async_copy(src_hbm.at[pl.ds(base, CHUNK)], src_v, sem3)
    cp4 = pltpu.async_copy(dst_hbm.at[pl.ds(base, CHUNK)], dst_v, sem4)
    cp1.wait()
    cp2.wait()
    cp3.wait()
    cp4.wait()

    @plsc.parallel_loop(0, FULL, unroll=2)
    def _body(i):
        sl = pl.ds(i * L, L)
        out_v[sl] = (plsc.load_gather(s1_v, [src_v[sl]])
                     + plsc.load_gather(s2_v, [dst_v[sl]]))

    tl = pl.ds(CHUNK - L, L)
    out_v[tl] = (plsc.load_gather(s1_v, [src_v[tl]])
                 + plsc.load_gather(s2_v, [dst_v[tl]]))

    pltpu.sync_copy(out_v, out_hbm.at[pl.ds(base, CHUNK)])


def kernel(z, edge_label_index, W, b):
    eli = edge_label_index.astype(jnp.int32)
    s, src, dst = pl.pallas_call(
        _tc_scores,
        grid=(GRID,),
        in_specs=[
            pl.BlockSpec((1, 2 * C), lambda i: (0, 0)),
            pl.BlockSpec((NBLK, C), lambda i: (i, 0)),
            pl.BlockSpec(memory_space=pltpu.SMEM),
            pl.BlockSpec((2, EBLK), lambda i: (0, i)),
        ],
        out_specs=[
            pl.BlockSpec((2, NBLK), lambda i: (0, i)),
            pl.BlockSpec((EBLK,), lambda i: (i,)),
            pl.BlockSpec((EBLK,), lambda i: (i,)),
        ],
        out_shape=[
            jax.ShapeDtypeStruct((2, N), jnp.float32),
            jax.ShapeDtypeStruct((E,), jnp.int32),
            jax.ShapeDtypeStruct((E,), jnp.int32),
        ],
    )(W, z, b, eli)

    return _sc_edge_scores(s, src, dst)


# speedup vs baseline: 1.0005x; 1.0005x over previous
import functools

import jax
import jax.numpy as jnp
from jax import lax
from jax.experimental import pallas as pl
from jax.experimental.pallas import tpu as pltpu
from jax.experimental.pallas import tpu_sc as plsc

C = 256
N = 10000
E = 160000

NC, NS, L = 2, 16, 16
NW = NC * NS
CHUNK = E // NW
FULL = CHUNK // L

GRID = 2
NBLK = 5120
EBLK = 81920


def _tc_scores(w_ref, z_ref, b_ref, eli_ref, s_ref, src_ref, dst_ref):
    w = w_ref[...]
    w12 = jnp.concatenate([w[:, :C], w[:, C:]], axis=0)
    s = lax.dot_general(
        w12, z_ref[...],
        dimension_numbers=(((1,), (1,)), ((), ())),
        preferred_element_type=jnp.float32,
    )
    row = lax.broadcasted_iota(jnp.int32, (2, NBLK), 0)
    s_ref[...] = s + jnp.where(row == 0, b_ref[0], 0.0)
    src_ref[...] = eli_ref[0]
    dst_ref[...] = eli_ref[1]


_sc_mesh = plsc.VectorSubcoreMesh(core_axis_name="c", subcore_axis_name="s")


@functools.partial(
    pl.kernel,
    out_type=jax.ShapeDtypeStruct((E,), jnp.float32),
    mesh=_sc_mesh,
    scratch_types=[
        pltpu.VMEM((N,), jnp.float32),
        pltpu.VMEM((N,), jnp.float32),
        pltpu.VMEM((CHUNK,), jnp.int32),
        pltpu.VMEM((CHUNK,), jnp.int32),
        pltpu.VMEM((CHUNK,), jnp.float32),
        pltpu.SemaphoreType.DMA,
        pltpu.SemaphoreType.DMA,
        pltpu.SemaphoreType.DMA,
        pltpu.SemaphoreType.DMA,
    ],
    compiler_params=pltpu.CompilerParams(needs_layout_passes=False),
)
def _sc_edge_scores(s_hbm, src_hbm, dst_hbm, out_hbm,
                    s1_v, s2_v, src_v, dst_v, out_v, sem1, sem2, sem3, sem4):
    wid = lax.axis_index("s") * NC + lax.axis_index("c")
    base = wid * CHUNK

    cp1 = pltpu.async_copy(s_hbm.at[0], s1_v, sem1)
    cp2 = pltpu.async_copy(s_hbm.at[1], s2_v, sem2)
    cp3 = pltpu.async_copy(src_hbm.at[pl.ds(base, CHUNK)], src_v, sem3)
    cp4 = pltpu.async_copy(dst_hbm.at[pl.ds(base, CHUNK)], dst_v, sem4)
    cp1.wait()
    cp2.wait()
    cp3.wait()
    cp4.wait()

    @plsc.parallel_loop(0, FULL, unroll=8)
    def _body(i):
        sl = pl.ds(i * L, L)
        out_v[sl] = (plsc.load_gather(s1_v, [src_v[sl]])
                     + plsc.load_gather(s2_v, [dst_v[sl]]))

    tl = pl.ds(CHUNK - L, L)
    out_v[tl] = (plsc.load_gather(s1_v, [src_v[tl]])
                 + plsc.load_gather(s2_v, [dst_v[tl]]))

    pltpu.sync_copy(out_v, out_hbm.at[pl.ds(base, CHUNK)])


def kernel(z, edge_label_index, W, b):
    eli = edge_label_index.astype(jnp.int32)
    s, src, dst = pl.pallas_call(
        _tc_scores,
        grid=(GRID,),
        in_specs=[
            pl.BlockSpec((1, 2 * C), lambda i: (0, 0)),
            pl.BlockSpec((NBLK, C), lambda i: (i, 0)),
            pl.BlockSpec(memory_space=pltpu.SMEM),
            pl.BlockSpec((2, EBLK), lambda i: (0, i)),
        ],
        out_specs=[
            pl.BlockSpec((2, NBLK), lambda i: (0, i)),
            pl.BlockSpec((EBLK,), lambda i: (i,)),
            pl.BlockSpec((EBLK,), lambda i: (i,)),
        ],
        out_shape=[
            jax.ShapeDtypeStruct((2, N), jnp.float32),
            jax.ShapeDtypeStruct((E,), jnp.int32),
            jax.ShapeDtypeStruct((E,), jnp.int32),
        ],
    )(W, z, b, eli)

    return _sc_edge_scores(s, src, dst)


# speedup vs baseline: 1.0160x; 1.0155x over previous
import functools

import numpy as np

import jax
import jax.numpy as jnp
from jax import lax
from jax.experimental import pallas as pl
from jax.experimental.pallas import tpu as pltpu
from jax.experimental.pallas import tpu_sc as plsc

C = 256
N = 10000
E = 160000

NC, NS, L = 2, 16, 16
NW = NC * NS
CHUNK = E // NW
FULL = CHUNK // L

GRID = 2
NBLK = 5120
EBLK = 81920

_HI = np.uint32(0xFFFF0000)
_RND = np.uint32(0x8000)

def _tc_scores(w_ref, z_ref, b_ref, eli_ref, t_ref, src_ref, dst_ref):
    w = w_ref[...]
    w12 = jnp.concatenate([w[:, :C], w[:, C:]], axis=0)
    s = lax.dot_general(
        w12, z_ref[...],
        dimension_numbers=(((1,), (1,)), ((), ())),
        preferred_element_type=jnp.float32,
    )
    row = lax.broadcasted_iota(jnp.int32, (2, NBLK), 0)
    s = s + jnp.where(row == 0, b_ref[0], 0.0)
    u = lax.bitcast_convert_type(s, jnp.uint32)
    u = u + _RND
    packed = (u[0] & _HI) | (u[1] >> 16)
    t_ref[...] = lax.bitcast_convert_type(packed, jnp.float32)
    src_ref[...] = eli_ref[0]
    dst_ref[...] = eli_ref[1]


_sc_mesh = plsc.VectorSubcoreMesh(core_axis_name="c", subcore_axis_name="s")


@functools.partial(
    pl.kernel,
    out_type=jax.ShapeDtypeStruct((E,), jnp.float32),
    mesh=_sc_mesh,
    scratch_types=[
        pltpu.VMEM((N,), jnp.float32),
        pltpu.VMEM((CHUNK,), jnp.int32),
        pltpu.VMEM((CHUNK,), jnp.int32),
        pltpu.VMEM((CHUNK,), jnp.float32),
        pltpu.SemaphoreType.DMA,
        pltpu.SemaphoreType.DMA,
        pltpu.SemaphoreType.DMA,
    ],
    compiler_params=pltpu.CompilerParams(needs_layout_passes=False),
)
def _sc_edge_scores(t_hbm, src_hbm, dst_hbm, out_hbm,
                    t_v, src_v, dst_v, out_v, sem1, sem2, sem3):
    wid = lax.axis_index("s") * NC + lax.axis_index("c")
    base = wid * CHUNK

    cp1 = pltpu.async_copy(t_hbm, t_v, sem1)
    cp2 = pltpu.async_copy(src_hbm.at[pl.ds(base, CHUNK)], src_v, sem2)
    cp3 = pltpu.async_copy(dst_hbm.at[pl.ds(base, CHUNK)], dst_v, sem3)
    cp1.wait()
    cp2.wait()

    tl = pl.ds(CHUNK - L, L)

    @plsc.parallel_loop(0, FULL, unroll=8)
    def _body1(i):
        sl = pl.ds(i * L, L)
        g = plsc.bitcast(plsc.load_gather(t_v, [src_v[sl]]), jnp.uint32)
        out_v[sl] = plsc.bitcast(g & _HI, jnp.float32)

    cp3.wait()

    @plsc.parallel_loop(0, FULL, unroll=8)
    def _body2(i):
        sl = pl.ds(i * L, L)
        g = plsc.bitcast(plsc.load_gather(t_v, [dst_v[sl]]), jnp.uint32)
        out_v[sl] = out_v[sl] + plsc.bitcast(g << 16, jnp.float32)

    g1 = plsc.bitcast(plsc.load_gather(t_v, [src_v[tl]]), jnp.uint32)
    g2 = plsc.bitcast(plsc.load_gather(t_v, [dst_v[tl]]), jnp.uint32)
    out_v[tl] = (plsc.bitcast(g1 & _HI, jnp.float32)
                 + plsc.bitcast(g2 << 16, jnp.float32))

    pltpu.sync_copy(out_v, out_hbm.at[pl.ds(base, CHUNK)])


def kernel(z, edge_label_index, W, b):
    eli = edge_label_index.astype(jnp.int32)
    t, src, dst = pl.pallas_call(
        _tc_scores,
        grid=(GRID,),
        in_specs=[
            pl.BlockSpec((1, 2 * C), lambda i: (0, 0)),
            pl.BlockSpec((NBLK, C), lambda i: (i, 0)),
            pl.BlockSpec(memory_space=pltpu.SMEM),
            pl.BlockSpec((2, EBLK), lambda i: (0, i)),
        ],
        out_specs=[
            pl.BlockSpec((NBLK,), lambda i: (i,)),
            pl.BlockSpec((EBLK,), lambda i: (i,)),
            pl.BlockSpec((EBLK,), lambda i: (i,)),
        ],
        out_shape=[
            jax.ShapeDtypeStruct((N,), jnp.float32),
            jax.ShapeDtypeStruct((E,), jnp.int32),
            jax.ShapeDtypeStruct((E,), jnp.int32),
        ],
    )(W, z, b, eli)

    return _sc_edge_scores(t, src, dst)


# speedup vs baseline: 1.0211x; 1.0050x over previous
import functools

import numpy as np

import jax
import jax.numpy as jnp
from jax import lax
from jax.experimental import pallas as pl
from jax.experimental.pallas import tpu as pltpu
from jax.experimental.pallas import tpu_sc as plsc

C = 256
N = 10000
E = 160000

NC, NS, L = 2, 16, 16
NW = NC * NS
CHUNK = E // NW
FULL = CHUNK // L

GRID = 2
NBLK = 5120
EBLK = 81920

_HI = np.uint32(0xFFFF0000)
_RND = np.uint32(0x8000)

def _tc_scores(w_ref, z_ref, b_ref, eli_ref, t_ref, src_ref, dst_ref):
    w = w_ref[...]
    w12 = jnp.concatenate([w[:, :C], w[:, C:]], axis=0)
    s = lax.dot_general(
        w12, z_ref[...],
        dimension_numbers=(((1,), (1,)), ((), ())),
        preferred_element_type=jnp.float32,
    )
    row = lax.broadcasted_iota(jnp.int32, (2, NBLK), 0)
    s = s + jnp.where(row == 0, b_ref[0], 0.0)
    u = lax.bitcast_convert_type(s, jnp.uint32)
    u = u + _RND
    packed = (u[0] & _HI) | (u[1] >> 16)
    t_ref[...] = lax.bitcast_convert_type(packed, jnp.float32)
    src_ref[...] = eli_ref[0]
    dst_ref[...] = eli_ref[1]


_sc_mesh = plsc.VectorSubcoreMesh(core_axis_name="c", subcore_axis_name="s")


@functools.partial(
    pl.kernel,
    out_type=jax.ShapeDtypeStruct((E,), jnp.float32),
    mesh=_sc_mesh,
    scratch_types=[
        pltpu.VMEM((N,), jnp.float32),
        pltpu.VMEM((CHUNK,), jnp.int32),
        pltpu.VMEM((CHUNK,), jnp.int32),
        pltpu.VMEM((CHUNK,), jnp.float32),
        pltpu.SemaphoreType.DMA,
        pltpu.SemaphoreType.DMA,
        pltpu.SemaphoreType.DMA,
    ],
    compiler_params=pltpu.CompilerParams(needs_layout_passes=False),
)
def _sc_edge_scores(t_hbm, src_hbm, dst_hbm, out_hbm,
                    t_v, src_v, dst_v, out_v, sem1, sem2, sem3):
    wid = lax.axis_index("s") * NC + lax.axis_index("c")
    base = wid * CHUNK

    cp1 = pltpu.async_copy(t_hbm, t_v, sem1)
    cp2 = pltpu.async_copy(src_hbm.at[pl.ds(base, CHUNK)], src_v, sem2)
    cp3 = pltpu.async_copy(dst_hbm.at[pl.ds(base, CHUNK)], dst_v, sem3)
    cp1.wait()
    cp2.wait()
    cp3.wait()

    tl = pl.ds(CHUNK - L, L)

    @plsc.parallel_loop(0, FULL, unroll=8)
    def _body(i):
        sl = pl.ds(i * L, L)
        g1 = plsc.bitcast(plsc.load_gather(t_v, [src_v[sl]]), jnp.uint32)
        g2 = plsc.bitcast(plsc.load_gather(t_v, [dst_v[sl]]), jnp.uint32)
        out_v[sl] = (plsc.bitcast(g1 & _HI, jnp.float32)
                     + plsc.bitcast(g2 << 16, jnp.float32))

    g1 = plsc.bitcast(plsc.load_gather(t_v, [src_v[tl]]), jnp.uint32)
    g2 = plsc.bitcast(plsc.load_gather(t_v, [dst_v[tl]]), jnp.uint32)
    out_v[tl] = (plsc.bitcast(g1 & _HI, jnp.float32)
                 + plsc.bitcast(g2 << 16, jnp.float32))

    pltpu.sync_copy(out_v, out_hbm.at[pl.ds(base, CHUNK)])


def kernel(z, edge_label_index, W, b):
    eli = edge_label_index.astype(jnp.int32)
    t, src, dst = pl.pallas_call(
        _tc_scores,
        grid=(GRID,),
        in_specs=[
            pl.BlockSpec((1, 2 * C), lambda i: (0, 0)),
            pl.BlockSpec((NBLK, C), lambda i: (i, 0)),
            pl.BlockSpec(memory_space=pltpu.SMEM),
            pl.BlockSpec((2, EBLK), lambda i: (0, i)),
        ],
        out_specs=[
            pl.BlockSpec((NBLK,), lambda i: (i,)),
            pl.BlockSpec((EBLK,), lambda i: (i,)),
            pl.BlockSpec((EBLK,), lambda i: (i,)),
        ],
        out_shape=[
            jax.ShapeDtypeStruct((N,), jnp.float32),
            jax.ShapeDtypeStruct((E,), jnp.int32),
            jax.ShapeDtypeStruct((E,), jnp.int32),
        ],
    )(W, z, b, eli)

    return _sc_edge_scores(t, src, dst)


# speedup vs baseline: 1.0455x; 1.0239x over previous
import functools

import numpy as np

import jax
import jax.numpy as jnp
from jax import lax
from jax.experimental import pallas as pl
from jax.experimental.pallas import tpu as pltpu
from jax.experimental.pallas import tpu_sc as plsc

C = 256
N = 10000
E = 160000

NC, NS, L = 2, 16, 16
NW = NC * NS
CHUNK = E // NW
FULL = CHUNK // L

GRID = 2
NBLK = 5120
EBLK = 81920

_HI = np.uint32(0xFFFF0000)
_RND = np.uint32(0x8000)
_LO14 = np.int32(0x3FFF)

def _tc_scores(w_ref, z_ref, b_ref, eli_ref, t_ref, idx_ref):
    w = w_ref[...]
    w12 = jnp.concatenate([w[:, :C], w[:, C:]], axis=0)
    s = lax.dot_general(
        w12, z_ref[...],
        dimension_numbers=(((1,), (1,)), ((), ())),
        preferred_element_type=jnp.float32,
    )
    row = lax.broadcasted_iota(jnp.int32, (2, NBLK), 0)
    s = s + jnp.where(row == 0, b_ref[0], 0.0)
    u = lax.bitcast_convert_type(s, jnp.uint32)
    u = u + _RND
    packed = (u[0] & _HI) | (u[1] >> 16)
    t_ref[...] = lax.bitcast_convert_type(packed, jnp.float32)
    idx_ref[...] = eli_ref[0] | (eli_ref[1] << 14)


_sc_mesh = plsc.VectorSubcoreMesh(core_axis_name="c", subcore_axis_name="s")


@functools.partial(
    pl.kernel,
    out_type=jax.ShapeDtypeStruct((E,), jnp.float32),
    mesh=_sc_mesh,
    scratch_types=[
        pltpu.VMEM((N,), jnp.float32),
        pltpu.VMEM((CHUNK,), jnp.int32),
        pltpu.VMEM((CHUNK,), jnp.float32),
        pltpu.SemaphoreType.DMA,
        pltpu.SemaphoreType.DMA,
    ],
    compiler_params=pltpu.CompilerParams(needs_layout_passes=False),
)
def _sc_edge_scores(t_hbm, idx_hbm, out_hbm,
                    t_v, idx_v, out_v, sem1, sem2):
    wid = lax.axis_index("s") * NC + lax.axis_index("c")
    base = wid * CHUNK

    cp1 = pltpu.async_copy(t_hbm, t_v, sem1)
    cp2 = pltpu.async_copy(idx_hbm.at[pl.ds(base, CHUNK)], idx_v, sem2)
    cp1.wait()
    cp2.wait()

    tl = pl.ds(CHUNK - L, L)

    @plsc.parallel_loop(0, FULL, unroll=8)
    def _body(i):
        sl = pl.ds(i * L, L)
        p = idx_v[sl]
        g1 = plsc.bitcast(plsc.load_gather(t_v, [p & _LO14]), jnp.uint32)
        g2 = plsc.bitcast(plsc.load_gather(t_v, [p >> 14]), jnp.uint32)
        out_v[sl] = (plsc.bitcast(g1 & _HI, jnp.float32)
                     + plsc.bitcast(g2 << 16, jnp.float32))

    p = idx_v[tl]
    g1 = plsc.bitcast(plsc.load_gather(t_v, [p & _LO14]), jnp.uint32)
    g2 = plsc.bitcast(plsc.load_gather(t_v, [p >> 14]), jnp.uint32)
    out_v[tl] = (plsc.bitcast(g1 & _HI, jnp.float32)
                 + plsc.bitcast(g2 << 16, jnp.float32))

    pltpu.sync_copy(out_v, out_hbm.at[pl.ds(base, CHUNK)])


def kernel(z, edge_label_index, W, b):
    eli = edge_label_index.astype(jnp.int32)
    t, idx = pl.pallas_call(
        _tc_scores,
        grid=(GRID,),
        in_specs=[
            pl.BlockSpec((1, 2 * C), lambda i: (0, 0)),
            pl.BlockSpec((NBLK, C), lambda i: (i, 0)),
            pl.BlockSpec(memory_space=pltpu.SMEM),
            pl.BlockSpec((2, EBLK), lambda i: (0, i)),
        ],
        out_specs=[
            pl.BlockSpec((NBLK,), lambda i: (i,)),
            pl.BlockSpec((EBLK,), lambda i: (i,)),
        ],
        out_shape=[
            jax.ShapeDtypeStruct((N,), jnp.float32),
            jax.ShapeDtypeStruct((E,), jnp.int32),
        ],
    )(W, z, b, eli)

    return _sc_edge_scores(t, idx)
